# Initial kernel scaffold; baseline (speedup 1.0000x reference)
import jax
import jax.numpy as jnp
from jax.experimental import pallas as pl


def kernel(node_feat, edge_index, edge_feat):
    raise NotImplementedError("write your pallas kernel here")


# speedup vs baseline: 4.0224x; 4.0224x over previous
import functools

import jax
import jax.numpy as jnp
from jax import lax
from jax.experimental import pallas as pl
from jax.experimental.pallas import tpu as pltpu
from jax.experimental.pallas import tpu_sc as plsc

NC = 2
NS = 16
LANES = 16
B = 128


def _sc_message_pass(N, D, E):
    nblk = E // B
    assert nblk * B == E
    nworkers = NC * NS
    nfull = nblk // nworkers
    nextra = nblk % nworkers
    ch = 80
    nch = N // ch
    assert nch * ch == N
    rounds = -(-nch // NS)

    mesh = plsc.VectorSubcoreMesh(core_axis_name="c", subcore_axis_name="s")

    @functools.partial(
        pl.kernel,
        mesh=mesh,
        out_type=jax.ShapeDtypeStruct((NC, N, D), jnp.float32),
        scratch_types=[
            pltpu.VMEM((B,), jnp.int32),
            pltpu.VMEM((B,), jnp.int32),
            pltpu.VMEM((B, D), jnp.float32),
            pltpu.VMEM((B, D), jnp.float32),
            pltpu.VMEM_SHARED((N, D), jnp.float32),
            pltpu.SemaphoreType.DMA,
        ],
    )
    def k(node_hbm, src_hbm, dst_hbm, edge_hbm, part_hbm,
          src_idx, dst_idx, m, g, acc, sem):
        cid = lax.axis_index("c")
        sid = lax.axis_index("s")
        wid = sid * NC + cid

        def zrow(r, _):
            for c in range(D // LANES):
                m[r, pl.ds(c * LANES, LANES)] = jnp.zeros((LANES,), jnp.float32)
            return 0
        lax.fori_loop(0, B, zrow, 0)
        for kk in range(rounds):
            j = kk * NS + sid
            @pl.when(j < nch)
            def _():
                pltpu.sync_copy(m.at[pl.ds(0, ch)], acc.at[pl.ds(j * ch, ch)])
        plsc.subcore_barrier()

        def do_block(blk):
            off = blk * B
            pltpu.sync_copy(src_hbm.at[pl.ds(off, B)], src_idx)
            pltpu.sync_copy(dst_hbm.at[pl.ds(off, B)], dst_idx)
            pltpu.sync_copy(edge_hbm.at[pl.ds(off, B)], m)
            pltpu.async_copy(node_hbm.at[src_idx], g, sem).wait()

            def row(r, _):
                for c in range(D // LANES):
                    sl = pl.ds(c * LANES, LANES)
                    m[r, sl] = jnp.maximum(m[r, sl] + g[r, sl], 0.0)
                return 0
            lax.fori_loop(0, B, row, 0)
            pltpu.sync_copy(m, acc.at[dst_idx], add=True)

        def blk_body(i, _):
            do_block(wid * nfull + i)
            return 0
        lax.fori_loop(0, nfull, blk_body, 0)
        if nextra:
            @pl.when(wid < nextra)
            def _():
                do_block(nworkers * nfull + wid)

        plsc.subcore_barrier()
        for kk in range(rounds):
            j = kk * NS + sid
            @pl.when(j < nch)
            def _():
                r0 = j * ch
                pltpu.sync_copy(acc.at[pl.ds(r0, ch)], m.at[pl.ds(0, ch)])
                pltpu.sync_copy(m.at[pl.ds(0, ch)],
                                part_hbm.at[cid, pl.ds(r0, ch)])

    return k


def _combine(x_ref, p_ref, o_ref):
    o_ref[...] = x_ref[...] + p_ref[0] + p_ref[1]


def kernel(node_feat, edge_index, edge_feat):
    N, D = node_feat.shape
    E = edge_feat.shape[0]
    src = edge_index[0]
    dst = edge_index[1]
    parts = _sc_message_pass(N, D, E)(node_feat, src, dst, edge_feat)

    rb = 1000 if N % 1000 == 0 else N
    out = pl.pallas_call(
        _combine,
        grid=(N // rb,),
        in_specs=[
            pl.BlockSpec((rb, D), lambda i: (i, 0)),
            pl.BlockSpec((NC, rb, D), lambda i: (0, i, 0)),
        ],
        out_specs=pl.BlockSpec((rb, D), lambda i: (i, 0)),
        out_shape=jax.ShapeDtypeStruct((N, D), jnp.float32),
    )(node_feat, parts)
    return out
